# baseline (speedup 1.0000x reference)
import functools

import jax
import jax.numpy as jnp
from jax import lax
from jax.experimental import pallas as pl
from jax.experimental.pallas import tpu as pltpu
from jax.experimental.pallas import tpu_sc as plsc

NC = 2
NS = 16
NW = NC * NS

B = 8192
D = 2048
E = 8
RB = 512
NRB = B // RB
TM = 256
NT = (B * 2 + E * TM) // TM
NPAD = NT * TM


def _router_body(x_ref, wr_ref, br_ref, gat_ref, auxi_ref, auxf_ref, meta_ref,
                 cnt0, off, cnt1, lg_ref):
    p = pl.program_id(0)
    b = pl.program_id(1)

    @pl.when(p == 0)
    def _():
        lg_ref[pl.ds(b * RB, RB), :] = jnp.dot(
            x_ref[...], wr_ref[...],
            preferred_element_type=jnp.float32) + br_ref[...]

    logits = lg_ref[pl.ds(b * RB, RB), :]
    lane_e = lax.broadcasted_iota(jnp.int32, (RB, E), 1)
    m0 = jnp.full((RB, 1), -jnp.inf, jnp.float32)
    i0 = jnp.zeros((RB, 1), jnp.int32)
    for e in range(E):
        c = logits[:, e:e + 1]
        upd = c > m0
        m0 = jnp.where(upd, c, m0)
        i0 = jnp.where(upd, e, i0)
    m1 = jnp.full((RB, 1), -jnp.inf, jnp.float32)
    i1 = jnp.zeros((RB, 1), jnp.int32)
    for e in range(E):
        c = logits[:, e:e + 1]
        upd = jnp.logical_and(c > m1, i0 != e)
        m1 = jnp.where(upd, c, m1)
        i1 = jnp.where(upd, e, i1)
    oh0 = (lane_e == i0).astype(jnp.float32)
    oh1 = (lane_e == i1).astype(jnp.float32)
    colsum0 = jnp.sum(oh0, axis=0, keepdims=True)
    colsum01 = colsum0 + jnp.sum(oh1, axis=0, keepdims=True)

    @pl.when(jnp.logical_and(p == 0, b == 0))
    def _():
        for e in range(E):
            cnt0[e] = 0

    @pl.when(p == 0)
    def _():
        for e in range(E):
            cnt0[e] = cnt0[e] + colsum01[0, e].astype(jnp.int32)

    @pl.when(jnp.logical_and(p == 0, b == NRB - 1))
    def _():
        acc = jnp.int32(0)
        for e in range(E):
            off[e] = acc
            acc = acc + ((cnt0[e] + TM - 1) // TM) * TM

    @pl.when(jnp.logical_and(p == 1, b == 0))
    def _():
        for e in range(E):
            cnt1[e] = 0

    @pl.when(p == 1)
    def _():
        t = jnp.exp(m1 - m0)
        w0v = 1.0 / (1.0 + t)
        w1v = t / (1.0 + t)
        gat_ref[...] = w0v * oh0 + w1v * oh1
        cv = jnp.zeros((1, E), jnp.float32)
        ov = jnp.zeros((1, E), jnp.float32)
        lane1 = lax.broadcasted_iota(jnp.int32, (1, E), 1)
        for e in range(E):
            cv = jnp.where(lane1 == e, cnt1[e].astype(jnp.float32), cv)
            ov = jnp.where(lane1 == e, off[e].astype(jnp.float32), ov)
        ii = lax.broadcasted_iota(jnp.int32, (RB, RB), 0)
        jj = lax.broadcasted_iota(jnp.int32, (RB, RB), 1)
        tri = (ii > jj).astype(jnp.float32)
        pre0 = jnp.dot(tri, oh0, preferred_element_type=jnp.float32)
        pre1 = jnp.dot(tri, oh1, preferred_element_type=jnp.float32) + colsum0
        base = ov + cv
        pos0 = jnp.sum((base + pre0) * oh0, axis=1, keepdims=True)
        pos1 = jnp.sum((base + pre1) * oh1, axis=1, keepdims=True)
        lane128 = lax.broadcasted_iota(jnp.int32, (RB, 128), 1)
        auxi_ref[...] = jnp.where(
            lane128 == 0, pos0.astype(jnp.int32),
            jnp.where(lane128 == 1, pos1.astype(jnp.int32), 0))
        auxf_ref[...] = jnp.where(lane128 == 0, w0v,
                                  jnp.where(lane128 == 1, w1v, 0.0))
        for e in range(E):
            cnt1[e] = cnt1[e] + colsum01[0, e].astype(jnp.int32)
        sub8 = lax.broadcasted_iota(jnp.int32, (E, 128), 0)
        mv = jnp.zeros((E, 128), jnp.int32)
        for e in range(E):
            mv = jnp.where(sub8 == e, off[e], mv)
        meta_ref[...] = mv


def _run_router(x, Wr, br):
    return pl.pallas_call(
        _router_body,
        grid=(2, NRB),
        in_specs=[
            pl.BlockSpec((RB, D), lambda p, b: (jnp.where(p == 0, b, 0), 0)),
            pl.BlockSpec((D, E), lambda p, b: (0, 0)),
            pl.BlockSpec((1, E), lambda p, b: (0, 0)),
        ],
        out_specs=[
            pl.BlockSpec((RB, E), lambda p, b: (b, 0)),
            pl.BlockSpec((RB, 128), lambda p, b: (b, 0)),
            pl.BlockSpec((RB, 128), lambda p, b: (b, 0)),
            pl.BlockSpec((E, 128), lambda p, b: (0, 0)),
        ],
        out_shape=[
            jax.ShapeDtypeStruct((B, E), jnp.float32),
            jax.ShapeDtypeStruct((B, 128), jnp.int32),
            jax.ShapeDtypeStruct((B, 128), jnp.float32),
            jax.ShapeDtypeStruct((E, 128), jnp.int32),
        ],
        scratch_shapes=[
            pltpu.SMEM((E,), jnp.int32),
            pltpu.SMEM((E,), jnp.int32),
            pltpu.SMEM((E,), jnp.int32),
            pltpu.VMEM((B, E), jnp.float32),
        ],
    )(x, Wr, br.reshape(1, E))


def _dense_body(x_ref, w1_ref, b1_ref, w2_ref, b2_ref, g_ref, o_ref):
    e = pl.program_id(1)
    xb = x_ref[...].astype(jnp.bfloat16)
    h = jnp.dot(xb, w1_ref[0], preferred_element_type=jnp.float32) + b1_ref[0]
    h = jnp.maximum(h, 0.0).astype(jnp.bfloat16)
    o = jnp.dot(h, w2_ref[0], preferred_element_type=jnp.float32) + b2_ref[0]
    lane_e = lax.broadcasted_iota(jnp.int32, (TM, E), 1)
    gcol = jnp.sum(jnp.where(lane_e == e, g_ref[...], 0.0), axis=1,
                   keepdims=True)
    val = o * gcol

    @pl.when(e == 0)
    def _():
        o_ref[...] = val

    @pl.when(e != 0)
    def _():
        o_ref[...] = o_ref[...] + val


def _run_dense(x, W1b, b1, W2b, b2, gat):
    nb = B // TM
    return pl.pallas_call(
        _dense_body,
        grid=(nb, E),
        in_specs=[
            pl.BlockSpec((TM, D), lambda b, e: (b, 0)),
            pl.BlockSpec((1, D, D), lambda b, e: (e, 0, 0)),
            pl.BlockSpec((1, 1, D), lambda b, e: (e, 0, 0)),
            pl.BlockSpec((1, D, D), lambda b, e: (e, 0, 0)),
            pl.BlockSpec((1, 1, D), lambda b, e: (e, 0, 0)),
            pl.BlockSpec((TM, E), lambda b, e: (b, 0)),
        ],
        out_specs=pl.BlockSpec((TM, D), lambda b, e: (b, 0)),
        out_shape=jax.ShapeDtypeStruct((B, D), jnp.float32),
    )(x, W1b, b1.reshape(E, 1, D), W2b, b2.reshape(E, 1, D), gat)


RPW = NPAD // NW
GCH = 32
TPW = B // NW
CCH = 16


def _dispatch_body(p0_hbm, p1_hbm, tok_hbm, p0v, p1v, tokv):
    wid = lax.axis_index("s") * NC + lax.axis_index("c")

    @pl.when(wid == 0)
    def _():
        pltpu.sync_copy(p0_hbm, p0v)
        pltpu.sync_copy(p1_hbm, p1v)
        zero = jnp.zeros((16,), jnp.int32)

        def zbody(i, carry):
            tokv[pl.ds(i * 16, 16)] = zero
            return carry

        lax.fori_loop(0, NPAD // 16, zbody, 0)

        def sbody(i, carry):
            toks = lax.broadcasted_iota(jnp.int32, (16,), 0) + i * 16
            plsc.store_scatter(tokv, [p0v[pl.ds(i * 16, 16)]], toks)
            plsc.store_scatter(tokv, [p1v[pl.ds(i * 16, 16)]], toks)
            return carry

        lax.fori_loop(0, B // 16, sbody, 0)
        pltpu.sync_copy(tokv, tok_hbm)


def _run_dispatch(pos0, pos1):
    return pl.kernel(
        _dispatch_body,
        out_type=jax.ShapeDtypeStruct((NPAD,), jnp.int32),
        mesh=plsc.VectorSubcoreMesh(core_axis_name="c", subcore_axis_name="s"),
        compiler_params=pltpu.CompilerParams(needs_layout_passes=False),
        scratch_types=[
            pltpu.VMEM((B,), jnp.int32),
            pltpu.VMEM((B,), jnp.int32),
            pltpu.VMEM((NPAD,), jnp.int32),
        ],
    )(pos0, pos1)


def _gather_body(x_hbm, tok_hbm, xs_hbm, idx0, idx1, idx2,
                 rows0, rows1, rows2, sg0, sg1, sg2, sc0, sc1, sc2):
    wid = lax.axis_index("s") * NC + lax.axis_index("c")
    base = wid * RPW
    n = RPW // GCH
    idxs = (idx0, idx1, idx2)
    rows = (rows0, rows1, rows2)
    sgs = (sg0, sg1, sg2)
    scs = (sc0, sc1, sc2)

    def issue(i):
        bb = i % 3
        pltpu.sync_copy(tok_hbm.at[pl.ds(base + i * GCH, GCH)], idxs[bb])
        return pltpu.async_copy(x_hbm.at[idxs[bb]], rows[bb], sgs[bb])

    g = [None] * n
    co = [None] * n
    g[0] = issue(0)
    g[1] = issue(1)
    for i in range(n):
        bb = i % 3
        g[i].wait()
        co[i] = pltpu.async_copy(
            rows[bb], xs_hbm.at[pl.ds(base + i * GCH, GCH)], scs[bb])
        if i >= 1:
            co[i - 1].wait()
        if i + 2 < n:
            g[i + 2] = issue(i + 2)
    co[n - 1].wait()


def _run_gather(x, tok):
    return pl.kernel(
        _gather_body,
        out_type=jax.ShapeDtypeStruct((NPAD, D // 2), jnp.int32),
        mesh=plsc.VectorSubcoreMesh(core_axis_name="c", subcore_axis_name="s"),
        scratch_types=[
            pltpu.VMEM((GCH,), jnp.int32),
            pltpu.VMEM((GCH,), jnp.int32),
            pltpu.VMEM((GCH,), jnp.int32),
            pltpu.VMEM((GCH, D // 2), jnp.int32),
            pltpu.VMEM((GCH, D // 2), jnp.int32),
            pltpu.VMEM((GCH, D // 2), jnp.int32),
            pltpu.SemaphoreType.DMA,
            pltpu.SemaphoreType.DMA,
            pltpu.SemaphoreType.DMA,
            pltpu.SemaphoreType.DMA,
            pltpu.SemaphoreType.DMA,
            pltpu.SemaphoreType.DMA,
        ],
    )(x, tok)


def _gemm_body(gm_ref, x_ref, w1_ref, b1_ref, w2_ref, b2_ref, o_ref):
    xb = x_ref[...]
    h = jnp.dot(xb, w1_ref[0], preferred_element_type=jnp.float32) + b1_ref[0]
    h = jnp.maximum(h, 0.0).astype(jnp.bfloat16)
    o_ref[...] = jnp.dot(h, w2_ref[0], preferred_element_type=jnp.float32) + b2_ref[0]


def _run_gemm(xs, W1b, b1, W2b, b2, gm):
    grid_spec = pltpu.PrefetchScalarGridSpec(
        num_scalar_prefetch=1,
        grid=(NT,),
        in_specs=[
            pl.BlockSpec((TM, D), lambda i, gm: (i, 0)),
            pl.BlockSpec((1, D, D), lambda i, gm: (gm[i], 0, 0)),
            pl.BlockSpec((1, 1, D), lambda i, gm: (gm[i], 0, 0)),
            pl.BlockSpec((1, D, D), lambda i, gm: (gm[i], 0, 0)),
            pl.BlockSpec((1, 1, D), lambda i, gm: (gm[i], 0, 0)),
        ],
        out_specs=pl.BlockSpec((TM, D), lambda i, gm: (i, 0)),
    )
    return pl.pallas_call(
        _gemm_body,
        grid_spec=grid_spec,
        out_shape=jax.ShapeDtypeStruct((NPAD, D), jnp.float32),
    )(gm, xs, W1b, b1.reshape(E, 1, D), W2b, b2.reshape(E, 1, D))


def _combine_body(y_hbm, p0_hbm, p1_hbm, w0_hbm, w1_hbm, out_hbm,
                  idxv, vav, vbv, w0v, w1v, sem):
    wid = lax.axis_index("s") * NC + lax.axis_index("c")
    tbase = wid * TPW
    pltpu.sync_copy(w0_hbm.at[pl.ds(tbase, TPW)], w0v)
    pltpu.sync_copy(w1_hbm.at[pl.ds(tbase, TPW)], w1v)

    def chunk(i, carry):
        off = tbase + i * CCH
        pltpu.sync_copy(p0_hbm.at[pl.ds(off, CCH)], idxv)
        pltpu.async_copy(y_hbm.at[idxv], vav, sem).wait()
        pltpu.sync_copy(p1_hbm.at[pl.ds(off, CCH)], idxv)
        pltpu.async_copy(y_hbm.at[idxv], vbv, sem).wait()
        for r in range(CCH):
            lidx = jnp.full((16,), i * CCH + r, jnp.int32)
            w0s = plsc.load_gather(w0v, [lidx])
            w1s = plsc.load_gather(w1v, [lidx])

            def kbody(kk, carry2, r=r, w0s=w0s, w1s=w1s):
                for j in range(8):
                    sl = pl.ds(kk * 128 + j * 16, 16)
                    vav[r, sl] = w0s * vav[r, sl] + w1s * vbv[r, sl]
                return carry2

            lax.fori_loop(0, D // 128, kbody, 0)
        pltpu.sync_copy(vav, out_hbm.at[pl.ds(off, CCH)])
        return carry

    lax.fori_loop(0, TPW // CCH, chunk, 0)


def _run_combine(Y, pos0, pos1, w0, w1):
    return pl.kernel(
        _combine_body,
        out_type=jax.ShapeDtypeStruct((B, D), jnp.float32),
        mesh=plsc.VectorSubcoreMesh(core_axis_name="c", subcore_axis_name="s"),
        compiler_params=pltpu.CompilerParams(needs_layout_passes=False),
        scratch_types=[
            pltpu.VMEM((CCH,), jnp.int32),
            pltpu.VMEM((CCH, D), jnp.float32),
            pltpu.VMEM((CCH, D), jnp.float32),
            pltpu.VMEM((TPW,), jnp.float32),
            pltpu.VMEM((TPW,), jnp.float32),
            pltpu.SemaphoreType.DMA,
        ],
    )(Y, pos0, pos1, w0, w1)


def kernel(x, Wr, br, W1, b1, W2, b2):
    x_bf = x.astype(jnp.bfloat16)
    Wrb = Wr.astype(jnp.bfloat16)
    gat, auxi, auxf, meta = _run_router(x_bf, Wrb, br)
    pos0 = auxi[:, 0]
    pos1 = auxi[:, 1]
    w0 = auxf[:, 0]
    w1 = auxf[:, 1]
    offp = meta[:, 0]
    tiles = jnp.arange(NT, dtype=jnp.int32) * TM
    gm = jnp.clip(jnp.sum((offp[None, :] <= tiles[:, None]).astype(jnp.int32),
                          axis=1) - 1, 0, E - 1).astype(jnp.int32)
    tok = _run_dispatch(pos0, pos1)
    x32 = lax.bitcast_convert_type(x_bf.reshape(B, D // 2, 2), jnp.int32)
    xs32 = _run_gather(x32, tok)
    xs = lax.bitcast_convert_type(xs32, jnp.bfloat16).reshape(NPAD, D)
    W1b = W1.astype(jnp.bfloat16)
    W2b = W2.astype(jnp.bfloat16)
    Y = _run_gemm(xs, W1b, b1, W2b, b2, gm)
    fused = _run_combine(Y, pos0, pos1, w0, w1)
    return fused, gat


# speedup vs baseline: 2.0428x; 2.0428x over previous
import functools

import jax
import jax.numpy as jnp
from jax import lax
from jax.experimental import pallas as pl
from jax.experimental.pallas import tpu as pltpu
from jax.experimental.pallas import tpu_sc as plsc

NC = 2
NS = 16
NW = NC * NS

B = 8192
D = 2048
E = 8
RB = 512
NRB = B // RB
TM = 256
NT = (B * 2 + E * TM) // TM
NPAD = NT * TM


def _router_body(x_ref, wr_ref, br_ref, gat_ref, auxi_ref, auxf_ref, meta_ref,
                 x32_ref, cnt0, off, cnt1, lg_ref):
    p = pl.program_id(0)
    b = pl.program_id(1)

    @pl.when(p == 0)
    def _():
        xb = x_ref[...].astype(jnp.bfloat16)
        wrb = wr_ref[...].astype(jnp.bfloat16)
        lg_ref[pl.ds(b * RB, RB), :] = jnp.dot(
            xb, wrb, preferred_element_type=jnp.float32) + br_ref[...]
        lo = lax.bitcast_convert_type(xb[:, :D // 2],
                                      jnp.uint16).astype(jnp.uint32)
        hi = lax.bitcast_convert_type(xb[:, D // 2:],
                                      jnp.uint16).astype(jnp.uint32)
        x32_ref[...] = lax.bitcast_convert_type((hi << 16) | lo, jnp.int32)

    logits = lg_ref[pl.ds(b * RB, RB), :]
    lane_e = lax.broadcasted_iota(jnp.int32, (RB, E), 1)
    m0 = jnp.full((RB, 1), -jnp.inf, jnp.float32)
    i0 = jnp.zeros((RB, 1), jnp.int32)
    for e in range(E):
        c = logits[:, e:e + 1]
        upd = c > m0
        m0 = jnp.where(upd, c, m0)
        i0 = jnp.where(upd, e, i0)
    m1 = jnp.full((RB, 1), -jnp.inf, jnp.float32)
    i1 = jnp.zeros((RB, 1), jnp.int32)
    for e in range(E):
        c = logits[:, e:e + 1]
        upd = jnp.logical_and(c > m1, i0 != e)
        m1 = jnp.where(upd, c, m1)
        i1 = jnp.where(upd, e, i1)
    oh0 = (lane_e == i0).astype(jnp.float32)
    oh1 = (lane_e == i1).astype(jnp.float32)
    colsum0 = jnp.sum(oh0, axis=0, keepdims=True)
    colsum01 = colsum0 + jnp.sum(oh1, axis=0, keepdims=True)

    @pl.when(jnp.logical_and(p == 0, b == 0))
    def _():
        for e in range(E):
            cnt0[e] = 0

    @pl.when(p == 0)
    def _():
        for e in range(E):
            cnt0[e] = cnt0[e] + colsum01[0, e].astype(jnp.int32)

    @pl.when(jnp.logical_and(p == 0, b == NRB - 1))
    def _():
        acc = jnp.int32(0)
        for e in range(E):
            off[e] = acc
            acc = acc + ((cnt0[e] + TM - 1) // TM) * TM

    @pl.when(jnp.logical_and(p == 1, b == 0))
    def _():
        for e in range(E):
            cnt1[e] = 0

    @pl.when(p == 1)
    def _():
        t = jnp.exp(m1 - m0)
        w0v = 1.0 / (1.0 + t)
        w1v = t / (1.0 + t)
        gat_ref[...] = w0v * oh0 + w1v * oh1
        cv = jnp.zeros((1, E), jnp.float32)
        ov = jnp.zeros((1, E), jnp.float32)
        lane1 = lax.broadcasted_iota(jnp.int32, (1, E), 1)
        for e in range(E):
            cv = jnp.where(lane1 == e, cnt1[e].astype(jnp.float32), cv)
            ov = jnp.where(lane1 == e, off[e].astype(jnp.float32), ov)
        ii = lax.broadcasted_iota(jnp.int32, (RB, RB), 0)
        jj = lax.broadcasted_iota(jnp.int32, (RB, RB), 1)
        tri = (ii > jj).astype(jnp.float32)
        pre0 = jnp.dot(tri, oh0, preferred_element_type=jnp.float32)
        pre1 = jnp.dot(tri, oh1, preferred_element_type=jnp.float32) + colsum0
        base = ov + cv
        pos0 = jnp.sum((base + pre0) * oh0, axis=1, keepdims=True)
        pos1 = jnp.sum((base + pre1) * oh1, axis=1, keepdims=True)
        auxi_ref[...] = jnp.where(
            lane_e == 0, pos0.astype(jnp.int32),
            jnp.where(lane_e == 1, pos1.astype(jnp.int32), 0))
        auxf_ref[...] = jnp.where(lane_e == 0, w0v,
                                  jnp.where(lane_e == 1, w1v, 0.0))
        for e in range(E):
            cnt1[e] = cnt1[e] + colsum01[0, e].astype(jnp.int32)
        sub8 = lax.broadcasted_iota(jnp.int32, (E, 128), 0)
        mv = jnp.zeros((E, 128), jnp.int32)
        for e in range(E):
            mv = jnp.where(sub8 == e, off[e], mv)
        meta_ref[...] = mv


def _run_router(x, Wr, br):
    return pl.pallas_call(
        _router_body,
        grid=(2, NRB),
        in_specs=[
            pl.BlockSpec((RB, D), lambda p, b: (jnp.where(p == 0, b, NRB - 1), 0)),
            pl.BlockSpec((D, E), lambda p, b: (0, 0)),
            pl.BlockSpec((1, E), lambda p, b: (0, 0)),
        ],
        out_specs=[
            pl.BlockSpec((RB, E), lambda p, b: (b, 0)),
            pl.BlockSpec((RB, E), lambda p, b: (b, 0)),
            pl.BlockSpec((RB, E), lambda p, b: (b, 0)),
            pl.BlockSpec((E, 128), lambda p, b: (0, 0)),
            pl.BlockSpec((RB, D // 2),
                         lambda p, b: (jnp.where(p == 0, b, NRB - 1), 0)),
        ],
        out_shape=[
            jax.ShapeDtypeStruct((B, E), jnp.float32),
            jax.ShapeDtypeStruct((B, E), jnp.int32),
            jax.ShapeDtypeStruct((B, E), jnp.float32),
            jax.ShapeDtypeStruct((E, 128), jnp.int32),
            jax.ShapeDtypeStruct((B, D // 2), jnp.int32),
        ],
        scratch_shapes=[
            pltpu.SMEM((E,), jnp.int32),
            pltpu.SMEM((E,), jnp.int32),
            pltpu.SMEM((E,), jnp.int32),
            pltpu.VMEM((B, E), jnp.float32),
        ],
    )(x, Wr, br.reshape(1, E))


def _dense_body(x_ref, w1_ref, b1_ref, w2_ref, b2_ref, g_ref, o_ref):
    e = pl.program_id(1)
    xb = x_ref[...].astype(jnp.bfloat16)
    h = jnp.dot(xb, w1_ref[0], preferred_element_type=jnp.float32) + b1_ref[0]
    h = jnp.maximum(h, 0.0).astype(jnp.bfloat16)
    o = jnp.dot(h, w2_ref[0], preferred_element_type=jnp.float32) + b2_ref[0]
    lane_e = lax.broadcasted_iota(jnp.int32, (TM, E), 1)
    gcol = jnp.sum(jnp.where(lane_e == e, g_ref[...], 0.0), axis=1,
                   keepdims=True)
    val = o * gcol

    @pl.when(e == 0)
    def _():
        o_ref[...] = val

    @pl.when(e != 0)
    def _():
        o_ref[...] = o_ref[...] + val


def _run_dense(x, W1b, b1, W2b, b2, gat):
    nb = B // TM
    return pl.pallas_call(
        _dense_body,
        grid=(nb, E),
        in_specs=[
            pl.BlockSpec((TM, D), lambda b, e: (b, 0)),
            pl.BlockSpec((1, D, D), lambda b, e: (e, 0, 0)),
            pl.BlockSpec((1, 1, D), lambda b, e: (e, 0, 0)),
            pl.BlockSpec((1, D, D), lambda b, e: (e, 0, 0)),
            pl.BlockSpec((1, 1, D), lambda b, e: (e, 0, 0)),
            pl.BlockSpec((TM, E), lambda b, e: (b, 0)),
        ],
        out_specs=pl.BlockSpec((TM, D), lambda b, e: (b, 0)),
        out_shape=jax.ShapeDtypeStruct((B, D), jnp.float32),
    )(x, W1b, b1.reshape(E, 1, D), W2b, b2.reshape(E, 1, D), gat)


RPW = NPAD // NW
GCH = 32
TPW = B // NW
CCH = 16


def _dispatch_body(p0_hbm, p1_hbm, tok_hbm, p0v, p1v, tokv):
    wid = lax.axis_index("s") * NC + lax.axis_index("c")

    @pl.when(wid == 0)
    def _():
        pltpu.sync_copy(p0_hbm, p0v)
        pltpu.sync_copy(p1_hbm, p1v)
        zero = jnp.zeros((16,), jnp.int32)

        def zbody(i, carry):
            tokv[pl.ds(i * 16, 16)] = zero
            return carry

        lax.fori_loop(0, NPAD // 16, zbody, 0)

        def sbody(i, carry):
            toks = lax.broadcasted_iota(jnp.int32, (16,), 0) + i * 16
            plsc.store_scatter(tokv, [p0v[pl.ds(i * 16, 16)]], toks)
            plsc.store_scatter(tokv, [p1v[pl.ds(i * 16, 16)]], toks)
            return carry

        lax.fori_loop(0, B // 16, sbody, 0)
        pltpu.sync_copy(tokv, tok_hbm)


def _run_dispatch(pos0, pos1):
    return pl.kernel(
        _dispatch_body,
        out_type=jax.ShapeDtypeStruct((NPAD,), jnp.int32),
        mesh=plsc.VectorSubcoreMesh(core_axis_name="c", subcore_axis_name="s"),
        compiler_params=pltpu.CompilerParams(needs_layout_passes=False),
        scratch_types=[
            pltpu.VMEM((B,), jnp.int32),
            pltpu.VMEM((B,), jnp.int32),
            pltpu.VMEM((NPAD,), jnp.int32),
        ],
    )(pos0, pos1)


def _gather_body(x_hbm, tok_hbm, xs_hbm, idx0, idx1, idx2,
                 rows0, rows1, rows2, sg0, sg1, sg2, sc0, sc1, sc2):
    wid = lax.axis_index("s") * NC + lax.axis_index("c")
    base = wid * RPW
    n = RPW // GCH
    idxs = (idx0, idx1, idx2)
    rows = (rows0, rows1, rows2)
    sgs = (sg0, sg1, sg2)
    scs = (sc0, sc1, sc2)

    def issue(i):
        bb = i % 3
        pltpu.sync_copy(tok_hbm.at[pl.ds(base + i * GCH, GCH)], idxs[bb])
        return pltpu.async_copy(x_hbm.at[idxs[bb]], rows[bb], sgs[bb])

    g = [None] * n
    co = [None] * n
    g[0] = issue(0)
    g[1] = issue(1)
    for i in range(n):
        bb = i % 3
        g[i].wait()
        co[i] = pltpu.async_copy(
            rows[bb], xs_hbm.at[pl.ds(base + i * GCH, GCH)], scs[bb])
        if i >= 1:
            co[i - 1].wait()
        if i + 2 < n:
            g[i + 2] = issue(i + 2)
    co[n - 1].wait()


def _run_gather(x, tok):
    return pl.kernel(
        _gather_body,
        out_type=jax.ShapeDtypeStruct((NPAD, D // 2), jnp.int32),
        mesh=plsc.VectorSubcoreMesh(core_axis_name="c", subcore_axis_name="s"),
        scratch_types=[
            pltpu.VMEM((GCH,), jnp.int32),
            pltpu.VMEM((GCH,), jnp.int32),
            pltpu.VMEM((GCH,), jnp.int32),
            pltpu.VMEM((GCH, D // 2), jnp.int32),
            pltpu.VMEM((GCH, D // 2), jnp.int32),
            pltpu.VMEM((GCH, D // 2), jnp.int32),
            pltpu.SemaphoreType.DMA,
            pltpu.SemaphoreType.DMA,
            pltpu.SemaphoreType.DMA,
            pltpu.SemaphoreType.DMA,
            pltpu.SemaphoreType.DMA,
            pltpu.SemaphoreType.DMA,
        ],
    )(x, tok)


def _gemm_body(gm_ref, x_ref, w1_ref, b1_ref, w2_ref, b2_ref, o_ref):
    xi = lax.bitcast_convert_type(x_ref[...], jnp.uint32)
    lo = lax.bitcast_convert_type((xi & 0xFFFF).astype(jnp.uint16),
                                  jnp.bfloat16)
    hi = lax.bitcast_convert_type((xi >> 16).astype(jnp.uint16),
                                  jnp.bfloat16)
    w1 = w1_ref[0]
    h = (jnp.dot(lo, w1[:D // 2, :], preferred_element_type=jnp.float32)
         + jnp.dot(hi, w1[D // 2:, :], preferred_element_type=jnp.float32)
         + b1_ref[0])
    h = jnp.maximum(h, 0.0).astype(jnp.bfloat16)
    o_ref[...] = jnp.dot(h, w2_ref[0], preferred_element_type=jnp.float32) + b2_ref[0]


def _run_gemm(xs, W1b, b1, W2b, b2, gm):
    grid_spec = pltpu.PrefetchScalarGridSpec(
        num_scalar_prefetch=1,
        grid=(NT,),
        in_specs=[
            pl.BlockSpec((TM, D // 2), lambda i, gm: (i, 0)),
            pl.BlockSpec((1, D, D), lambda i, gm: (gm[i], 0, 0)),
            pl.BlockSpec((1, 1, D), lambda i, gm: (gm[i], 0, 0)),
            pl.BlockSpec((1, D, D), lambda i, gm: (gm[i], 0, 0)),
            pl.BlockSpec((1, 1, D), lambda i, gm: (gm[i], 0, 0)),
        ],
        out_specs=pl.BlockSpec((TM, D), lambda i, gm: (i, 0)),
    )
    return pl.pallas_call(
        _gemm_body,
        grid_spec=grid_spec,
        out_shape=jax.ShapeDtypeStruct((NPAD, D), jnp.float32),
    )(gm, xs, W1b, b1.reshape(E, 1, D), W2b, b2.reshape(E, 1, D))


def _combine_body(y_hbm, p0_hbm, p1_hbm, w0_hbm, w1_hbm, out_hbm,
                  idxv, vav, vbv, w0v, w1v, sem):
    wid = lax.axis_index("s") * NC + lax.axis_index("c")
    tbase = wid * TPW
    pltpu.sync_copy(w0_hbm.at[pl.ds(tbase, TPW)], w0v)
    pltpu.sync_copy(w1_hbm.at[pl.ds(tbase, TPW)], w1v)

    def chunk(i, carry):
        off = tbase + i * CCH
        pltpu.sync_copy(p0_hbm.at[pl.ds(off, CCH)], idxv)
        pltpu.async_copy(y_hbm.at[idxv], vav, sem).wait()
        pltpu.sync_copy(p1_hbm.at[pl.ds(off, CCH)], idxv)
        pltpu.async_copy(y_hbm.at[idxv], vbv, sem).wait()
        for r in range(CCH):
            lidx = jnp.full((16,), i * CCH + r, jnp.int32)
            w0s = plsc.load_gather(w0v, [lidx])
            w1s = plsc.load_gather(w1v, [lidx])

            def kbody(kk, carry2, r=r, w0s=w0s, w1s=w1s):
                for j in range(8):
                    sl = pl.ds(kk * 128 + j * 16, 16)
                    vav[r, sl] = w0s * vav[r, sl] + w1s * vbv[r, sl]
                return carry2

            lax.fori_loop(0, D // 128, kbody, 0)
        pltpu.sync_copy(vav, out_hbm.at[pl.ds(off, CCH)])
        return carry

    lax.fori_loop(0, TPW // CCH, chunk, 0)


def _run_combine(Y, pos0, pos1, w0, w1):
    return pl.kernel(
        _combine_body,
        out_type=jax.ShapeDtypeStruct((B, D), jnp.float32),
        mesh=plsc.VectorSubcoreMesh(core_axis_name="c", subcore_axis_name="s"),
        compiler_params=pltpu.CompilerParams(needs_layout_passes=False),
        scratch_types=[
            pltpu.VMEM((CCH,), jnp.int32),
            pltpu.VMEM((CCH, D), jnp.float32),
            pltpu.VMEM((CCH, D), jnp.float32),
            pltpu.VMEM((TPW,), jnp.float32),
            pltpu.VMEM((TPW,), jnp.float32),
            pltpu.SemaphoreType.DMA,
        ],
    )(Y, pos0, pos1, w0, w1)


def kernel(x, Wr, br, W1, b1, W2, b2):
    gat, auxi, auxf, meta, x32 = _run_router(x, Wr, br)
    pos0 = auxi[:, 0]
    pos1 = auxi[:, 1]
    w0 = auxf[:, 0]
    w1 = auxf[:, 1]
    offp = meta[:, 0]
    tiles = jnp.arange(NT, dtype=jnp.int32) * TM
    gm = jnp.clip(jnp.sum((offp[None, :] <= tiles[:, None]).astype(jnp.int32),
                          axis=1) - 1, 0, E - 1).astype(jnp.int32)
    tok = _run_dispatch(pos0, pos1)
    xs32 = _run_gather(x32, tok)
    W1b = W1.astype(jnp.bfloat16)
    W2b = W2.astype(jnp.bfloat16)
    Y = _run_gemm(xs32, W1b, b1, W2b, b2, gm)
    fused = _run_combine(Y, pos0, pos1, w0, w1)
    return fused, gat


# speedup vs baseline: 2.0995x; 1.0278x over previous
import functools

import jax
import jax.numpy as jnp
from jax import lax
from jax.experimental import pallas as pl
from jax.experimental.pallas import tpu as pltpu
from jax.experimental.pallas import tpu_sc as plsc

NC = 2
NS = 16
NW = NC * NS

B = 8192
D = 2048
E = 8
RB = 512
NRB = B // RB
TM = 256
NT = (B * 2 + E * TM) // TM
NPAD = NT * TM


def _router_body(x_ref, wr_ref, br_ref, gat_ref, auxi_ref, auxf_ref, meta_ref,
                 x32_ref, cnt0, off, cnt1, lg_ref):
    p = pl.program_id(0)
    b = pl.program_id(1)

    @pl.when(p == 0)
    def _():
        xb = x_ref[...].astype(jnp.bfloat16)
        wrb = wr_ref[...].astype(jnp.bfloat16)
        lg_ref[pl.ds(b * RB, RB), :] = jnp.dot(
            xb, wrb, preferred_element_type=jnp.float32) + br_ref[...]
        lo = lax.bitcast_convert_type(xb[:, :D // 2],
                                      jnp.uint16).astype(jnp.uint32)
        hi = lax.bitcast_convert_type(xb[:, D // 2:],
                                      jnp.uint16).astype(jnp.uint32)
        x32_ref[...] = lax.bitcast_convert_type((hi << 16) | lo, jnp.int32)

    logits = lg_ref[pl.ds(b * RB, RB), :]
    lane_e = lax.broadcasted_iota(jnp.int32, (RB, E), 1)
    m0 = jnp.full((RB, 1), -jnp.inf, jnp.float32)
    i0 = jnp.zeros((RB, 1), jnp.int32)
    for e in range(E):
        c = logits[:, e:e + 1]
        upd = c > m0
        m0 = jnp.where(upd, c, m0)
        i0 = jnp.where(upd, e, i0)
    m1 = jnp.full((RB, 1), -jnp.inf, jnp.float32)
    i1 = jnp.zeros((RB, 1), jnp.int32)
    for e in range(E):
        c = logits[:, e:e + 1]
        upd = jnp.logical_and(c > m1, i0 != e)
        m1 = jnp.where(upd, c, m1)
        i1 = jnp.where(upd, e, i1)
    oh0 = (lane_e == i0).astype(jnp.float32)
    oh1 = (lane_e == i1).astype(jnp.float32)
    colsum0 = jnp.sum(oh0, axis=0, keepdims=True)
    colsum01 = colsum0 + jnp.sum(oh1, axis=0, keepdims=True)

    @pl.when(jnp.logical_and(p == 0, b == 0))
    def _():
        for e in range(E):
            cnt0[e] = 0

    @pl.when(p == 0)
    def _():
        for e in range(E):
            cnt0[e] = cnt0[e] + colsum01[0, e].astype(jnp.int32)

    @pl.when(jnp.logical_and(p == 0, b == NRB - 1))
    def _():
        acc = jnp.int32(0)
        for e in range(E):
            off[e] = acc
            acc = acc + ((cnt0[e] + TM - 1) // TM) * TM

    @pl.when(jnp.logical_and(p == 1, b == 0))
    def _():
        for e in range(E):
            cnt1[e] = 0

    @pl.when(p == 1)
    def _():
        t = jnp.exp(m1 - m0)
        w0v = 1.0 / (1.0 + t)
        w1v = t / (1.0 + t)
        gat_ref[...] = w0v * oh0 + w1v * oh1
        cv = jnp.zeros((1, E), jnp.float32)
        ov = jnp.zeros((1, E), jnp.float32)
        lane1 = lax.broadcasted_iota(jnp.int32, (1, E), 1)
        for e in range(E):
            cv = jnp.where(lane1 == e, cnt1[e].astype(jnp.float32), cv)
            ov = jnp.where(lane1 == e, off[e].astype(jnp.float32), ov)
        ii = lax.broadcasted_iota(jnp.int32, (RB, RB), 0)
        jj = lax.broadcasted_iota(jnp.int32, (RB, RB), 1)
        tri = (ii > jj).astype(jnp.float32)
        pre0 = jnp.dot(tri, oh0, preferred_element_type=jnp.float32)
        pre1 = jnp.dot(tri, oh1, preferred_element_type=jnp.float32) + colsum0
        base = ov + cv
        pos0 = jnp.sum((base + pre0) * oh0, axis=1, keepdims=True)
        pos1 = jnp.sum((base + pre1) * oh1, axis=1, keepdims=True)
        auxi_ref[...] = jnp.where(
            lane_e == 0, pos0.astype(jnp.int32),
            jnp.where(lane_e == 1, pos1.astype(jnp.int32), 0))
        auxf_ref[...] = jnp.where(lane_e == 0, w0v,
                                  jnp.where(lane_e == 1, w1v, 0.0))
        for e in range(E):
            cnt1[e] = cnt1[e] + colsum01[0, e].astype(jnp.int32)
        sub8 = lax.broadcasted_iota(jnp.int32, (E, 128), 0)
        mv = jnp.zeros((E, 128), jnp.int32)
        for e in range(E):
            mv = jnp.where(sub8 == e, off[e], mv)
        meta_ref[...] = mv


def _run_router(x, Wr, br):
    return pl.pallas_call(
        _router_body,
        grid=(2, NRB),
        in_specs=[
            pl.BlockSpec((RB, D), lambda p, b: (jnp.where(p == 0, b, NRB - 1), 0)),
            pl.BlockSpec((D, E), lambda p, b: (0, 0)),
            pl.BlockSpec((1, E), lambda p, b: (0, 0)),
        ],
        out_specs=[
            pl.BlockSpec((RB, E), lambda p, b: (b, 0)),
            pl.BlockSpec((RB, E), lambda p, b: (b, 0)),
            pl.BlockSpec((RB, E), lambda p, b: (b, 0)),
            pl.BlockSpec((E, 128), lambda p, b: (0, 0)),
            pl.BlockSpec((RB, D // 2),
                         lambda p, b: (jnp.where(p == 0, b, NRB - 1), 0)),
        ],
        out_shape=[
            jax.ShapeDtypeStruct((B, E), jnp.float32),
            jax.ShapeDtypeStruct((B, E), jnp.int32),
            jax.ShapeDtypeStruct((B, E), jnp.float32),
            jax.ShapeDtypeStruct((E, 128), jnp.int32),
            jax.ShapeDtypeStruct((B, D // 2), jnp.int32),
        ],
        scratch_shapes=[
            pltpu.SMEM((E,), jnp.int32),
            pltpu.SMEM((E,), jnp.int32),
            pltpu.SMEM((E,), jnp.int32),
            pltpu.VMEM((B, E), jnp.float32),
        ],
    )(x, Wr, br.reshape(1, E))


def _dense_body(x_ref, w1_ref, b1_ref, w2_ref, b2_ref, g_ref, o_ref):
    e = pl.program_id(1)
    xb = x_ref[...].astype(jnp.bfloat16)
    h = jnp.dot(xb, w1_ref[0], preferred_element_type=jnp.float32) + b1_ref[0]
    h = jnp.maximum(h, 0.0).astype(jnp.bfloat16)
    o = jnp.dot(h, w2_ref[0], preferred_element_type=jnp.float32) + b2_ref[0]
    lane_e = lax.broadcasted_iota(jnp.int32, (TM, E), 1)
    gcol = jnp.sum(jnp.where(lane_e == e, g_ref[...], 0.0), axis=1,
                   keepdims=True)
    val = o * gcol

    @pl.when(e == 0)
    def _():
        o_ref[...] = val

    @pl.when(e != 0)
    def _():
        o_ref[...] = o_ref[...] + val


def _run_dense(x, W1b, b1, W2b, b2, gat):
    nb = B // TM
    return pl.pallas_call(
        _dense_body,
        grid=(nb, E),
        in_specs=[
            pl.BlockSpec((TM, D), lambda b, e: (b, 0)),
            pl.BlockSpec((1, D, D), lambda b, e: (e, 0, 0)),
            pl.BlockSpec((1, 1, D), lambda b, e: (e, 0, 0)),
            pl.BlockSpec((1, D, D), lambda b, e: (e, 0, 0)),
            pl.BlockSpec((1, 1, D), lambda b, e: (e, 0, 0)),
            pl.BlockSpec((TM, E), lambda b, e: (b, 0)),
        ],
        out_specs=pl.BlockSpec((TM, D), lambda b, e: (b, 0)),
        out_shape=jax.ShapeDtypeStruct((B, D), jnp.float32),
    )(x, W1b, b1.reshape(E, 1, D), W2b, b2.reshape(E, 1, D), gat)


GCH = 24
TPW = B // NW
CCH = 16


def _dispatch_body(p0_hbm, p1_hbm, tok_hbm, p0v, p1v, tokv):
    wid = lax.axis_index("s") * NC + lax.axis_index("c")

    @pl.when(wid == 0)
    def _():
        pltpu.sync_copy(p0_hbm, p0v)
        pltpu.sync_copy(p1_hbm, p1v)
        zero = jnp.zeros((16,), jnp.int32)

        def zbody(i, carry):
            tokv[pl.ds(i * 16, 16)] = zero
            return carry

        lax.fori_loop(0, NPAD // 16, zbody, 0)

        def sbody(i, carry):
            toks = lax.broadcasted_iota(jnp.int32, (16,), 0) + i * 16
            plsc.store_scatter(tokv, [p0v[pl.ds(i * 16, 16)]], toks)
            plsc.store_scatter(tokv, [p1v[pl.ds(i * 16, 16)]], toks)
            return carry

        lax.fori_loop(0, B // 16, sbody, 0)
        pltpu.sync_copy(tokv, tok_hbm)


def _run_dispatch(pos0, pos1):
    return pl.kernel(
        _dispatch_body,
        out_type=jax.ShapeDtypeStruct((NPAD,), jnp.int32),
        mesh=plsc.VectorSubcoreMesh(core_axis_name="c", subcore_axis_name="s"),
        compiler_params=pltpu.CompilerParams(needs_layout_passes=False),
        scratch_types=[
            pltpu.VMEM((B,), jnp.int32),
            pltpu.VMEM((B,), jnp.int32),
            pltpu.VMEM((NPAD,), jnp.int32),
        ],
    )(pos0, pos1)


def _gather_body(qoff, x_hbm, tok_hbm, xs_hbm, idx0, idx1, idx2,
                 rows0, rows1, rows2, sg0, sg1, sg2, sc0, sc1, sc2):
    wid = lax.axis_index("s") * NC + lax.axis_index("c")
    base = wid * RPWQ
    n = RPWQ // GCH
    idxs = (idx0, idx1, idx2)
    rows = (rows0, rows1, rows2)
    sgs = (sg0, sg1, sg2)
    scs = (sc0, sc1, sc2)

    def issue(i):
        bb = i % 3
        pltpu.sync_copy(tok_hbm.at[pl.ds(qoff + base + i * GCH, GCH)],
                        idxs[bb])
        return pltpu.async_copy(x_hbm.at[idxs[bb]], rows[bb], sgs[bb])

    g = [None] * n
    co = [None] * n
    g[0] = issue(0)
    g[1] = issue(1)
    for i in range(n):
        bb = i % 3
        g[i].wait()
        co[i] = pltpu.async_copy(
            rows[bb], xs_hbm.at[pl.ds(base + i * GCH, GCH)], scs[bb])
        if i >= 1:
            co[i - 1].wait()
        if i + 2 < n:
            g[i + 2] = issue(i + 2)
    co[n - 1].wait()


def _run_gather(x, tok, q):
    return pl.kernel(
        functools.partial(_gather_body, q * QROWS),
        out_type=jax.ShapeDtypeStruct((QROWS, D // 2), jnp.int32),
        mesh=plsc.VectorSubcoreMesh(core_axis_name="c", subcore_axis_name="s"),
        scratch_types=[
            pltpu.VMEM((GCH,), jnp.int32),
            pltpu.VMEM((GCH,), jnp.int32),
            pltpu.VMEM((GCH,), jnp.int32),
            pltpu.VMEM((GCH, D // 2), jnp.int32),
            pltpu.VMEM((GCH, D // 2), jnp.int32),
            pltpu.VMEM((GCH, D // 2), jnp.int32),
            pltpu.SemaphoreType.DMA,
            pltpu.SemaphoreType.DMA,
            pltpu.SemaphoreType.DMA,
            pltpu.SemaphoreType.DMA,
            pltpu.SemaphoreType.DMA,
            pltpu.SemaphoreType.DMA,
        ],
    )(x, tok)


NSPL = 4
NTQ = NT // NSPL
QROWS = NTQ * TM
RPWQ = QROWS // NW


def _gemm_body(gm_ref, x_ref, w1_ref, b1_ref, w2_ref, b2_ref, *rest):
    o_ref = rest[-1]
    xi = lax.bitcast_convert_type(x_ref[...], jnp.uint32)
    lo = lax.bitcast_convert_type((xi & 0xFFFF).astype(jnp.uint16),
                                  jnp.bfloat16)
    hi = lax.bitcast_convert_type((xi >> 16).astype(jnp.uint16),
                                  jnp.bfloat16)
    w1 = w1_ref[0]
    h = (jnp.dot(lo, w1[:D // 2, :], preferred_element_type=jnp.float32)
         + jnp.dot(hi, w1[D // 2:, :], preferred_element_type=jnp.float32)
         + b1_ref[0])
    h = jnp.maximum(h, 0.0).astype(jnp.bfloat16)
    o_ref[...] = jnp.dot(h, w2_ref[0],
                         preferred_element_type=jnp.float32) + b2_ref[0]


def _run_gemm(xs_q, W1b, b1, W2b, b2, gm, q, y_in):
    in_specs = [
        pl.BlockSpec((TM, D // 2), lambda i, gm: (i, 0)),
        pl.BlockSpec((1, D, D), lambda i, gm: (gm[q * NTQ + i], 0, 0)),
        pl.BlockSpec((1, 1, D), lambda i, gm: (gm[q * NTQ + i], 0, 0)),
        pl.BlockSpec((1, D, D), lambda i, gm: (gm[q * NTQ + i], 0, 0)),
        pl.BlockSpec((1, 1, D), lambda i, gm: (gm[q * NTQ + i], 0, 0)),
    ]
    args = [gm, xs_q, W1b, b1.reshape(E, 1, D), W2b, b2.reshape(E, 1, D)]
    aliases = {}
    if y_in is not None:
        in_specs.append(pl.BlockSpec(memory_space=pl.ANY))
        args.append(y_in)
        aliases = {6: 0}
    grid_spec = pltpu.PrefetchScalarGridSpec(
        num_scalar_prefetch=1,
        grid=(NTQ,),
        in_specs=in_specs,
        out_specs=pl.BlockSpec((TM, D), lambda i, gm: (q * NTQ + i, 0)),
    )
    return pl.pallas_call(
        _gemm_body,
        grid_spec=grid_spec,
        out_shape=jax.ShapeDtypeStruct((NPAD, D), jnp.float32),
        input_output_aliases=aliases,
    )(*args)


def _combine_body(y_hbm, p0_hbm, p1_hbm, w0_hbm, w1_hbm, out_hbm,
                  idxv, vav, vbv, w0v, w1v, sem):
    wid = lax.axis_index("s") * NC + lax.axis_index("c")
    tbase = wid * TPW
    pltpu.sync_copy(w0_hbm.at[pl.ds(tbase, TPW)], w0v)
    pltpu.sync_copy(w1_hbm.at[pl.ds(tbase, TPW)], w1v)

    def chunk(i, carry):
        off = tbase + i * CCH
        pltpu.sync_copy(p0_hbm.at[pl.ds(off, CCH)], idxv)
        pltpu.async_copy(y_hbm.at[idxv], vav, sem).wait()
        pltpu.sync_copy(p1_hbm.at[pl.ds(off, CCH)], idxv)
        pltpu.async_copy(y_hbm.at[idxv], vbv, sem).wait()
        for r in range(CCH):
            lidx = jnp.full((16,), i * CCH + r, jnp.int32)
            w0s = plsc.load_gather(w0v, [lidx])
            w1s = plsc.load_gather(w1v, [lidx])

            def kbody(kk, carry2, r=r, w0s=w0s, w1s=w1s):
                for j in range(8):
                    sl = pl.ds(kk * 128 + j * 16, 16)
                    vav[r, sl] = w0s * vav[r, sl] + w1s * vbv[r, sl]
                return carry2

            lax.fori_loop(0, D // 128, kbody, 0)
        pltpu.sync_copy(vav, out_hbm.at[pl.ds(off, CCH)])
        return carry

    lax.fori_loop(0, TPW // CCH, chunk, 0)


def _run_combine(Y, pos0, pos1, w0, w1):
    return pl.kernel(
        _combine_body,
        out_type=jax.ShapeDtypeStruct((B, D), jnp.float32),
        mesh=plsc.VectorSubcoreMesh(core_axis_name="c", subcore_axis_name="s"),
        compiler_params=pltpu.CompilerParams(needs_layout_passes=False),
        scratch_types=[
            pltpu.VMEM((CCH,), jnp.int32),
            pltpu.VMEM((CCH, D), jnp.float32),
            pltpu.VMEM((CCH, D), jnp.float32),
            pltpu.VMEM((TPW,), jnp.float32),
            pltpu.VMEM((TPW,), jnp.float32),
            pltpu.SemaphoreType.DMA,
        ],
    )(Y, pos0, pos1, w0, w1)


def kernel(x, Wr, br, W1, b1, W2, b2):
    gat, auxi, auxf, meta, x32 = _run_router(x, Wr, br)
    pos0 = auxi[:, 0]
    pos1 = auxi[:, 1]
    w0 = auxf[:, 0]
    w1 = auxf[:, 1]
    offp = meta[:, 0]
    tiles = jnp.arange(NT, dtype=jnp.int32) * TM
    gm = jnp.clip(jnp.sum((offp[None, :] <= tiles[:, None]).astype(jnp.int32),
                          axis=1) - 1, 0, E - 1).astype(jnp.int32)
    tok = _run_dispatch(pos0, pos1)
    W1b = W1.astype(jnp.bfloat16)
    W2b = W2.astype(jnp.bfloat16)
    Y = None
    for q in range(NSPL):
        xs_q = _run_gather(x32, tok, q)
        Y = _run_gemm(xs_q, W1b, b1, W2b, b2, gm, q, Y)
    fused = _run_combine(Y, pos0, pos1, w0, w1)
    return fused, gat
